# initial kernel scaffold (unmeasured)
import jax
import jax.numpy as jnp
from jax import lax
from jax.experimental import pallas as pl
from jax.experimental.pallas import tpu as pltpu

T = 4096
V_PER = 8192
D = 2048
HALF = T // 2


def kernel(ids, E):
    my_x = lax.axis_index("x")
    my_y = lax.axis_index("y")

    my_ids = lax.dynamic_slice(ids, (my_y * HALF,), (HALF,))
    local = my_ids - my_x * V_PER
    mask = (local >= 0) & (local < V_PER)
    safe = jnp.where(mask, local, 0)
    partial = jnp.where(mask[:, None], E[safe], 0.0).astype(jnp.bfloat16)

    def body(partial_ref, out_ref, xrecv, sum_buf, yrecv,
             sem_xs, sem_xr, sem_ys, sem_yr):
        x = lax.axis_index("x")
        y = lax.axis_index("y")

        barrier = pltpu.get_barrier_semaphore()
        pl.semaphore_signal(barrier, inc=1, device_id=(1 - x, y),
                            device_id_type=pl.DeviceIdType.MESH)
        pl.semaphore_signal(barrier, inc=1, device_id=(x, 1 - y),
                            device_id_type=pl.DeviceIdType.MESH)
        pl.semaphore_wait(barrier, 2)

        rdma_x = pltpu.make_async_remote_copy(
            src_ref=partial_ref, dst_ref=xrecv,
            send_sem=sem_xs, recv_sem=sem_xr,
            device_id=(1 - x, y), device_id_type=pl.DeviceIdType.MESH)
        rdma_x.start()
        rdma_x.wait()

        sum_buf[...] = partial_ref[...] + xrecv[...]

        rdma_y = pltpu.make_async_remote_copy(
            src_ref=sum_buf, dst_ref=yrecv,
            send_sem=sem_ys, recv_sem=sem_yr,
            device_id=(x, 1 - y), device_id_type=pl.DeviceIdType.MESH)
        rdma_y.start()
        rdma_y.wait()

        out_ref[pl.ds(y * HALF, HALF), :] = sum_buf[...].astype(jnp.float32)
        out_ref[pl.ds((1 - y) * HALF, HALF), :] = yrecv[...].astype(jnp.float32)

    return pl.pallas_call(
        body,
        out_shape=jax.ShapeDtypeStruct((T, D), jnp.float32),
        in_specs=[pl.BlockSpec(memory_space=pltpu.VMEM)],
        out_specs=pl.BlockSpec(memory_space=pltpu.VMEM),
        scratch_shapes=[
            pltpu.VMEM((HALF, D), jnp.bfloat16),
            pltpu.VMEM((HALF, D), jnp.bfloat16),
            pltpu.VMEM((HALF, D), jnp.bfloat16),
            pltpu.SemaphoreType.DMA,
            pltpu.SemaphoreType.DMA,
            pltpu.SemaphoreType.DMA,
            pltpu.SemaphoreType.DMA,
        ],
        compiler_params=pltpu.CompilerParams(collective_id=0),
    )(partial)


# baseline (device time: 290439 ns/iter reference)
import jax
import jax.numpy as jnp
from jax import lax
from jax.experimental import pallas as pl
from jax.experimental.pallas import tpu as pltpu

T = 4096
V_PER = 8192
D = 2048
HALF = T // 2


def kernel(ids, E):
    my_x = lax.axis_index("x")
    my_y = lax.axis_index("y")

    my_ids = lax.dynamic_slice(ids, (my_y * HALF,), (HALF,))
    local = my_ids - my_x * V_PER
    mask = (local >= 0) & (local < V_PER)
    safe = jnp.where(mask, local, 0)
    partial = jnp.where(mask[:, None], E[safe], 0.0).astype(jnp.bfloat16)

    def body(partial_ref, out_ref, xrecv,
             sem_xs, sem_xr, sem_ys, sem_yr):
        x = lax.axis_index("x")
        y = lax.axis_index("y")

        barrier = pltpu.get_barrier_semaphore()
        pl.semaphore_signal(barrier, inc=1, device_id=(1 - x, y),
                            device_id_type=pl.DeviceIdType.MESH)
        pl.semaphore_signal(barrier, inc=1, device_id=(x, 1 - y),
                            device_id_type=pl.DeviceIdType.MESH)
        pl.semaphore_wait(barrier, 2)

        rdma_x = pltpu.make_async_remote_copy(
            src_ref=partial_ref, dst_ref=xrecv,
            send_sem=sem_xs, recv_sem=sem_xr,
            device_id=(1 - x, y), device_id_type=pl.DeviceIdType.MESH)
        rdma_x.start()
        rdma_x.wait()

        out_ref[pl.ds(y * HALF, HALF), :] = partial_ref[...] + xrecv[...]

        rdma_y = pltpu.make_async_remote_copy(
            src_ref=out_ref.at[pl.ds(y * HALF, HALF)],
            dst_ref=out_ref.at[pl.ds(y * HALF, HALF)],
            send_sem=sem_ys, recv_sem=sem_yr,
            device_id=(x, 1 - y), device_id_type=pl.DeviceIdType.MESH)
        rdma_y.start()
        rdma_y.wait()

    return pl.pallas_call(
        body,
        out_shape=jax.ShapeDtypeStruct((T, D), jnp.bfloat16),
        in_specs=[pl.BlockSpec(memory_space=pltpu.VMEM)],
        out_specs=pl.BlockSpec(memory_space=pltpu.VMEM),
        scratch_shapes=[
            pltpu.VMEM((HALF, D), jnp.bfloat16),
            pltpu.SemaphoreType.DMA,
            pltpu.SemaphoreType.DMA,
            pltpu.SemaphoreType.DMA,
            pltpu.SemaphoreType.DMA,
        ],
        compiler_params=pltpu.CompilerParams(
            collective_id=0, vmem_limit_bytes=100 * 1024 * 1024
        ),
    )(partial)


# device time: 175439 ns/iter; 1.6555x vs baseline; 1.6555x over previous
import jax
import jax.numpy as jnp
from jax import lax
from jax.experimental import pallas as pl
from jax.experimental.pallas import tpu as pltpu

T = 4096
V_PER = 8192
D = 2048
HALF = T // 2
C = 16
CH = HALF // C


def kernel(ids, E):
    my_x = lax.axis_index("x")
    my_y = lax.axis_index("y")

    my_ids = lax.dynamic_slice(ids, (my_y * HALF,), (HALF,))
    local = my_ids - my_x * V_PER
    mask = ((local >= 0) & (local < V_PER))
    safe = jnp.where(mask, local, 0).astype(jnp.int32)
    mask_f = mask.astype(jnp.float32)[:, None]

    def body(sid_ref, mask_ref, e_ref, out_ref,
             gbuf, partial, xrecv,
             gsem, xsem_s, xsem_r, ysem_s, ysem_r):
        x = lax.axis_index("x")
        y = lax.axis_index("y")

        def issue_gather(c):
            def row(i, _):
                idx = sid_ref[c * CH + i]
                pltpu.make_async_copy(
                    e_ref.at[pl.ds(idx, 1)],
                    gbuf.at[c % 2, pl.ds(i, 1)],
                    gsem.at[c % 2],
                ).start()
                return 0
            lax.fori_loop(0, CH, row, 0)

        def wait_gather(c):
            def row(i, _):
                pltpu.make_async_copy(
                    e_ref.at[pl.ds(0, 1)],
                    gbuf.at[c % 2, pl.ds(i, 1)],
                    gsem.at[c % 2],
                ).wait()
                return 0
            lax.fori_loop(0, CH, row, 0)

        def rdma_x(c):
            return pltpu.make_async_remote_copy(
                src_ref=partial.at[pl.ds(c * CH, CH)],
                dst_ref=xrecv.at[pl.ds(c * CH, CH)],
                send_sem=xsem_s.at[c], recv_sem=xsem_r.at[c],
                device_id=(1 - x, y), device_id_type=pl.DeviceIdType.MESH)

        def rdma_y(c):
            return pltpu.make_async_remote_copy(
                src_ref=out_ref.at[pl.ds(y * HALF + c * CH, CH)],
                dst_ref=out_ref.at[pl.ds(y * HALF + c * CH, CH)],
                send_sem=ysem_s.at[c], recv_sem=ysem_r.at[c],
                device_id=(x, 1 - y), device_id_type=pl.DeviceIdType.MESH)

        issue_gather(0)
        issue_gather(1)

        barrier = pltpu.get_barrier_semaphore()
        pl.semaphore_signal(barrier, inc=1, device_id=(1 - x, y),
                            device_id_type=pl.DeviceIdType.MESH)
        pl.semaphore_signal(barrier, inc=1, device_id=(x, 1 - y),
                            device_id_type=pl.DeviceIdType.MESH)
        pl.semaphore_wait(barrier, 2)

        for c in range(C):
            wait_gather(c)
            partial[pl.ds(c * CH, CH), :] = gbuf[c % 2].astype(jnp.bfloat16)
            rdma_x(c).start()
            if c + 2 < C:
                issue_gather(c + 2)

        for c in range(C):
            rdma_x(c).wait()
            m = mask_ref[pl.ds(c * CH, CH), :] > 0.0
            out_ref[pl.ds(y * HALF + c * CH, CH), :] = jnp.where(
                m, partial[pl.ds(c * CH, CH), :], xrecv[pl.ds(c * CH, CH), :])
            rdma_y(c).start()

        for c in range(C):
            rdma_y(c).wait()

    return pl.pallas_call(
        body,
        out_shape=jax.ShapeDtypeStruct((T, D), jnp.bfloat16),
        in_specs=[
            pl.BlockSpec(memory_space=pltpu.SMEM),
            pl.BlockSpec(memory_space=pltpu.VMEM),
            pl.BlockSpec(memory_space=pl.ANY),
        ],
        out_specs=pl.BlockSpec(memory_space=pltpu.VMEM),
        scratch_shapes=[
            pltpu.VMEM((2, CH, D), jnp.float32),
            pltpu.VMEM((HALF, D), jnp.bfloat16),
            pltpu.VMEM((HALF, D), jnp.bfloat16),
            pltpu.SemaphoreType.DMA((2,)),
            pltpu.SemaphoreType.DMA((C,)),
            pltpu.SemaphoreType.DMA((C,)),
            pltpu.SemaphoreType.DMA((C,)),
            pltpu.SemaphoreType.DMA((C,)),
        ],
        compiler_params=pltpu.CompilerParams(
            collective_id=0, vmem_limit_bytes=100 * 1024 * 1024
        ),
    )(safe, mask_f, E)


# device time: 162975 ns/iter; 1.7821x vs baseline; 1.0765x over previous
import jax
import jax.numpy as jnp
from jax import lax
from jax.experimental import pallas as pl
from jax.experimental.pallas import tpu as pltpu

T = 4096
V_PER = 8192
D = 2048
HALF = T // 2
C = 16
CH = HALF // C


def kernel(ids, E):
    my_x = lax.axis_index("x")
    my_y = lax.axis_index("y")

    my_ids = lax.dynamic_slice(ids, (my_y * HALF,), (HALF,))
    local = my_ids - my_x * V_PER
    mask = ((local >= 0) & (local < V_PER))
    safe = jnp.where(mask, local, 0).astype(jnp.int32)
    mask_f = mask.astype(jnp.float32)[:, None]

    def body(sid_ref, mask_ref, e_ref, out_ref,
             gbuf, partial, xrecv,
             gsem, xsem_s, xsem_r, ysem_s, ysem_r):
        x = lax.axis_index("x")
        y = lax.axis_index("y")

        def issue_gather(c):
            def row(i, _):
                idx = sid_ref[c * CH + i]
                pltpu.make_async_copy(
                    e_ref.at[pl.ds(idx, 1)],
                    gbuf.at[c % 2, pl.ds(i, 1)],
                    gsem.at[c % 2],
                ).start()
                return 0
            lax.fori_loop(0, CH, row, 0)

        def wait_gather(c):
            pltpu.make_async_copy(
                e_ref.at[pl.ds(0, CH)],
                gbuf.at[c % 2],
                gsem.at[c % 2],
            ).wait()

        def rdma_x(c):
            return pltpu.make_async_remote_copy(
                src_ref=partial.at[pl.ds(c * CH, CH)],
                dst_ref=xrecv.at[pl.ds(c * CH, CH)],
                send_sem=xsem_s.at[c], recv_sem=xsem_r.at[c],
                device_id=(1 - x, y), device_id_type=pl.DeviceIdType.MESH)

        def rdma_y(c):
            return pltpu.make_async_remote_copy(
                src_ref=out_ref.at[pl.ds(y * HALF + c * CH, CH)],
                dst_ref=out_ref.at[pl.ds(y * HALF + c * CH, CH)],
                send_sem=ysem_s.at[c], recv_sem=ysem_r.at[c],
                device_id=(x, 1 - y), device_id_type=pl.DeviceIdType.MESH)

        issue_gather(0)
        issue_gather(1)

        barrier = pltpu.get_barrier_semaphore()
        pl.semaphore_signal(barrier, inc=1, device_id=(1 - x, y),
                            device_id_type=pl.DeviceIdType.MESH)
        pl.semaphore_signal(barrier, inc=1, device_id=(x, 1 - y),
                            device_id_type=pl.DeviceIdType.MESH)
        pl.semaphore_wait(barrier, 2)

        for c in range(C):
            wait_gather(c)
            partial[pl.ds(c * CH, CH), :] = gbuf[c % 2].astype(jnp.bfloat16)
            rdma_x(c).start()
            if c + 2 < C:
                issue_gather(c + 2)

        for c in range(C):
            rdma_x(c).wait()
            m = mask_ref[pl.ds(c * CH, CH), :] > 0.0
            out_ref[pl.ds(y * HALF + c * CH, CH), :] = jnp.where(
                m, partial[pl.ds(c * CH, CH), :], xrecv[pl.ds(c * CH, CH), :])
            rdma_y(c).start()

        for c in range(C):
            rdma_y(c).wait()

    return pl.pallas_call(
        body,
        out_shape=jax.ShapeDtypeStruct((T, D), jnp.bfloat16),
        in_specs=[
            pl.BlockSpec(memory_space=pltpu.SMEM),
            pl.BlockSpec(memory_space=pltpu.VMEM),
            pl.BlockSpec(memory_space=pl.ANY),
        ],
        out_specs=pl.BlockSpec(memory_space=pltpu.VMEM),
        scratch_shapes=[
            pltpu.VMEM((2, CH, D), jnp.float32),
            pltpu.VMEM((HALF, D), jnp.bfloat16),
            pltpu.VMEM((HALF, D), jnp.bfloat16),
            pltpu.SemaphoreType.DMA((2,)),
            pltpu.SemaphoreType.DMA((C,)),
            pltpu.SemaphoreType.DMA((C,)),
            pltpu.SemaphoreType.DMA((C,)),
            pltpu.SemaphoreType.DMA((C,)),
        ],
        compiler_params=pltpu.CompilerParams(
            collective_id=0, vmem_limit_bytes=100 * 1024 * 1024
        ),
    )(safe, mask_f, E)


# device time: 132904 ns/iter; 2.1853x vs baseline; 1.2263x over previous
import jax
import jax.numpy as jnp
from jax import lax
from jax.experimental import pallas as pl
from jax.experimental.pallas import tpu as pltpu

T = 4096
V_PER = 8192
D = 2048
HALF = T // 2
C = 16
CH = HALF // C


def kernel(ids, E):
    my_x = lax.axis_index("x")
    my_y = lax.axis_index("y")

    my_ids = lax.dynamic_slice(ids, (my_y * HALF,), (HALF,))
    local = my_ids - my_x * V_PER
    mask = ((local >= 0) & (local < V_PER))
    safe = jnp.where(mask, local, 0).astype(jnp.int32)
    mask_f = mask.astype(jnp.float32)[:, None]

    def body(sid_ref, mask_ref, e_ref, out_ref,
             gbuf, partial, xrecv,
             gsem, xsem_s, xsem_r, ysem_s, ysem_r):
        x = lax.axis_index("x")
        y = lax.axis_index("y")

        def issue_gather(c):
            def row(i, _):
                idx = sid_ref[c * CH + i]
                pltpu.make_async_copy(
                    e_ref.at[pl.ds(idx, 1)],
                    gbuf.at[c % 2, pl.ds(i, 1)],
                    gsem.at[c % 2],
                ).start()
                return 0
            lax.fori_loop(0, CH, row, 0)

        def wait_gather(c):
            pltpu.make_async_copy(
                e_ref.at[pl.ds(0, CH)],
                gbuf.at[c % 2],
                gsem.at[c % 2],
            ).wait()

        def rdma_x(c):
            return pltpu.make_async_remote_copy(
                src_ref=partial.at[pl.ds(c * CH, CH)],
                dst_ref=xrecv.at[pl.ds(c * CH, CH)],
                send_sem=xsem_s.at[c], recv_sem=xsem_r.at[c],
                device_id=(1 - x, y), device_id_type=pl.DeviceIdType.MESH)

        def rdma_y(c):
            return pltpu.make_async_remote_copy(
                src_ref=out_ref.at[pl.ds(y * HALF + c * CH, CH)],
                dst_ref=out_ref.at[pl.ds(y * HALF + c * CH, CH)],
                send_sem=ysem_s.at[c], recv_sem=ysem_r.at[c],
                device_id=(x, 1 - y), device_id_type=pl.DeviceIdType.MESH)

        issue_gather(0)
        issue_gather(1)

        barrier = pltpu.get_barrier_semaphore()
        pl.semaphore_signal(barrier, inc=1, device_id=(1 - x, y),
                            device_id_type=pl.DeviceIdType.MESH)
        pl.semaphore_signal(barrier, inc=1, device_id=(x, 1 - y),
                            device_id_type=pl.DeviceIdType.MESH)
        pl.semaphore_wait(barrier, 2)

        def merge_and_forward(d):
            rdma_x(d).wait()
            m = mask_ref[pl.ds(d * CH, CH), :] > 0.0
            out_ref[pl.ds(y * HALF + d * CH, CH), :] = jnp.where(
                m, partial[pl.ds(d * CH, CH), :], xrecv[pl.ds(d * CH, CH), :])
            rdma_y(d).start()

        LAG = 2
        for c in range(C):
            wait_gather(c)
            partial[pl.ds(c * CH, CH), :] = gbuf[c % 2].astype(jnp.bfloat16)
            rdma_x(c).start()
            if c + 2 < C:
                issue_gather(c + 2)
            if c >= LAG:
                merge_and_forward(c - LAG)
        for d in range(C - LAG, C):
            merge_and_forward(d)

        for c in range(C):
            rdma_y(c).wait()

    return pl.pallas_call(
        body,
        out_shape=jax.ShapeDtypeStruct((T, D), jnp.bfloat16),
        in_specs=[
            pl.BlockSpec(memory_space=pltpu.SMEM),
            pl.BlockSpec(memory_space=pltpu.VMEM),
            pl.BlockSpec(memory_space=pl.ANY),
        ],
        out_specs=pl.BlockSpec(memory_space=pltpu.VMEM),
        scratch_shapes=[
            pltpu.VMEM((2, CH, D), jnp.float32),
            pltpu.VMEM((HALF, D), jnp.bfloat16),
            pltpu.VMEM((HALF, D), jnp.bfloat16),
            pltpu.SemaphoreType.DMA((2,)),
            pltpu.SemaphoreType.DMA((C,)),
            pltpu.SemaphoreType.DMA((C,)),
            pltpu.SemaphoreType.DMA((C,)),
            pltpu.SemaphoreType.DMA((C,)),
        ],
        compiler_params=pltpu.CompilerParams(
            collective_id=0, vmem_limit_bytes=100 * 1024 * 1024
        ),
    )(safe, mask_f, E)


# device time: 124931 ns/iter; 2.3248x vs baseline; 1.0638x over previous
import jax
import jax.numpy as jnp
from jax import lax
from jax.experimental import pallas as pl
from jax.experimental.pallas import tpu as pltpu

T = 4096
V_PER = 8192
D = 2048
HALF = T // 2
C = 64
CH = HALF // C


def kernel(ids, E):
    my_x = lax.axis_index("x")
    my_y = lax.axis_index("y")

    my_ids = lax.dynamic_slice(ids, (my_y * HALF,), (HALF,))
    local = my_ids - my_x * V_PER
    mask = ((local >= 0) & (local < V_PER))
    safe = jnp.where(mask, local, 0).astype(jnp.int32)
    mask_f = mask.astype(jnp.float32)[:, None]

    def body(sid_ref, mask_ref, e_ref, out_ref,
             gbuf, partial, xrecv,
             gsem, xsem_s, xsem_r, ysem_s, ysem_r):
        x = lax.axis_index("x")
        y = lax.axis_index("y")

        def issue_gather(c):
            def row(i, _):
                idx = sid_ref[c * CH + i]
                pltpu.make_async_copy(
                    e_ref.at[pl.ds(idx, 1)],
                    gbuf.at[c % 2, pl.ds(i, 1)],
                    gsem.at[c % 2],
                ).start()
                return 0
            lax.fori_loop(0, CH, row, 0, unroll=8)

        def wait_gather(c):
            pltpu.make_async_copy(
                e_ref.at[pl.ds(0, CH)],
                gbuf.at[c % 2],
                gsem.at[c % 2],
            ).wait()

        def rdma_x(c):
            return pltpu.make_async_remote_copy(
                src_ref=partial.at[pl.ds(c * CH, CH)],
                dst_ref=xrecv.at[pl.ds(c * CH, CH)],
                send_sem=xsem_s.at[c], recv_sem=xsem_r.at[c],
                device_id=(1 - x, y), device_id_type=pl.DeviceIdType.MESH)

        def rdma_y(c):
            return pltpu.make_async_remote_copy(
                src_ref=out_ref.at[pl.ds(y * HALF + c * CH, CH)],
                dst_ref=out_ref.at[pl.ds(y * HALF + c * CH, CH)],
                send_sem=ysem_s.at[c], recv_sem=ysem_r.at[c],
                device_id=(x, 1 - y), device_id_type=pl.DeviceIdType.MESH)

        issue_gather(0)
        issue_gather(1)

        barrier = pltpu.get_barrier_semaphore()
        pl.semaphore_signal(barrier, inc=1, device_id=(1 - x, y),
                            device_id_type=pl.DeviceIdType.MESH)
        pl.semaphore_signal(barrier, inc=1, device_id=(x, 1 - y),
                            device_id_type=pl.DeviceIdType.MESH)
        pl.semaphore_wait(barrier, 2)

        def merge_and_forward(d):
            rdma_x(d).wait()
            m = mask_ref[pl.ds(d * CH, CH), :] > 0.0
            out_ref[pl.ds(y * HALF + d * CH, CH), :] = jnp.where(
                m, partial[pl.ds(d * CH, CH), :], xrecv[pl.ds(d * CH, CH), :])
            rdma_y(d).start()

        LAG = 2
        for c in range(C):
            wait_gather(c)
            partial[pl.ds(c * CH, CH), :] = gbuf[c % 2].astype(jnp.bfloat16)
            rdma_x(c).start()
            if c + 2 < C:
                issue_gather(c + 2)
            if c >= LAG:
                merge_and_forward(c - LAG)
        for d in range(C - LAG, C):
            merge_and_forward(d)

        for c in range(C):
            rdma_y(c).wait()

    return pl.pallas_call(
        body,
        out_shape=jax.ShapeDtypeStruct((T, D), jnp.bfloat16),
        in_specs=[
            pl.BlockSpec(memory_space=pltpu.SMEM),
            pl.BlockSpec(memory_space=pltpu.VMEM),
            pl.BlockSpec(memory_space=pl.ANY),
        ],
        out_specs=pl.BlockSpec(memory_space=pltpu.VMEM),
        scratch_shapes=[
            pltpu.VMEM((2, CH, D), jnp.float32),
            pltpu.VMEM((HALF, D), jnp.bfloat16),
            pltpu.VMEM((HALF, D), jnp.bfloat16),
            pltpu.SemaphoreType.DMA((2,)),
            pltpu.SemaphoreType.DMA((C,)),
            pltpu.SemaphoreType.DMA((C,)),
            pltpu.SemaphoreType.DMA((C,)),
            pltpu.SemaphoreType.DMA((C,)),
        ],
        compiler_params=pltpu.CompilerParams(
            collective_id=0, vmem_limit_bytes=100 * 1024 * 1024
        ),
    )(safe, mask_f, E)
